# baseline (speedup 1.0000x reference)
import functools
import math
import random as _random

import numpy as np
import jax
import jax.numpy as jnp
from jax import lax
from jax.experimental import pallas as pl
from jax.experimental.pallas import tpu as pltpu
from jax.experimental.pallas import tpu_sc as plsc

_NUM_HASHES = 2
_VOCAB = 1000000
_DIM = 32
_NUM_EMBEDS = 100000
_BATCH = 16384

_NC = 2
_NS = 16
_NW = _NC * _NS
_BPW = _BATCH // _NW
_CHUNK = 128
_NCHUNK = _BPW // _CHUNK

_MIN32 = -0x80000000
_MAGIC = 0x5F3759DF


def _draw_hash_constants():
    def is_prime(x):
        for i in range(2, int(math.sqrt(x))):
            if x % i == 0:
                return False
        return True

    def next_prime(n):
        while not is_prime(n):
            n += 1
        return n

    rng = _random.Random()
    rng.seed(1924031)

    def draw(N):
        p = next_prime(rng.randint(_VOCAB, int(2 ** 32)))
        a, b = rng.randint(1, p), rng.randint(1, p)
        return (a, b, p, N)

    hs = [draw(_NUM_EMBEDS) for _ in range(_NUM_HASHES)]
    wh = draw(_VOCAB)
    return hs + [wh]


_HASH_SPECS = _draw_hash_constants()


def _build_tables():
    tabs = np.zeros((len(_HASH_SPECS) * 2048,), dtype=np.uint32)
    for h, (a, b, p, _N) in enumerate(_HASH_SPECS):
        j = np.arange(1024, dtype=object)
        t1 = np.array([(a * 1024 * int(x)) % p for x in range(1024)], dtype=np.uint64)
        t0 = np.array([(a * int(x) + b) % p for x in range(1024)], dtype=np.uint64)
        tabs[h * 2048: h * 2048 + 1024] = t1.astype(np.uint32)
        tabs[h * 2048 + 1024: h * 2048 + 2048] = t0.astype(np.uint32)
    return jnp.asarray(tabs.view(np.int32))


_TABLES = _build_tables()


def _i32(v):
    return np.int32(np.uint32(v & 0xFFFFFFFF)).item()


def _hash_mod(tabs_ref, x, h):
    a, b, p, N = _HASH_SPECS[h]
    off = h * 2048
    x1 = lax.shift_right_logical(x, jnp.full(x.shape, 10, jnp.int32))
    x0 = jnp.bitwise_and(x, 1023)
    t1 = plsc.load_gather(tabs_ref, [x1 + off])
    t0 = plsc.load_gather(tabs_ref, [x0 + (off + 1024)])
    s = t1 + t0
    sm = jnp.bitwise_xor(s, _MIN32)
    t1m = jnp.bitwise_xor(t1, _MIN32)
    cond = jnp.logical_or(sm < t1m, sm >= _i32(p ^ 0x80000000))
    r = jnp.where(cond, s - _i32(p), s)
    hi = lax.shift_right_logical(r, jnp.full(r.shape, 8, jnp.int32))
    lo = jnp.bitwise_and(r, 255)
    rf = lax.convert_element_type(hi, jnp.float32) * 256.0 + \
        lax.convert_element_type(lo, jnp.float32)
    q = lax.convert_element_type(rf * (1.0 / N), jnp.int32)
    rem = r - q * N
    rem = jnp.where(rem < 0, rem + N, rem)
    rem = jnp.where(rem >= N, rem - N, rem)
    return rem


def _body(ids_hbm, tabs_hbm, emb_hbm, wemb_hbm, lnw_hbm, lnb_hbm, out_hbm,
          tabs_v, ids_v, idx0_v, idx1_v, widx_v, wrow_v, e0_v, e1_v, wg_v,
          lnw_v, lnb_v, out_v, sem):
    wid = lax.axis_index("s") * jnp.int32(_NC) + lax.axis_index("c")
    base = wid * jnp.int32(_BPW)

    pltpu.sync_copy(ids_hbm.at[pl.ds(base, _BPW)], ids_v)
    pltpu.sync_copy(tabs_hbm, tabs_v)
    pltpu.sync_copy(lnw_hbm, lnw_v)
    pltpu.sync_copy(lnb_hbm, lnb_v)

    def hash_row(r, _):
        for gg in range(_CHUNK // 16):
            x = ids_v[pl.ds(r * jnp.int32(_CHUNK) + jnp.int32(gg * 16), 16)]
            idx0_v[r, pl.ds(gg * 16, 16)] = _hash_mod(tabs_v, x, 0)
            idx1_v[r, pl.ds(gg * 16, 16)] = _hash_mod(tabs_v, x, 1)
            wh = _hash_mod(tabs_v, x, 2)
            widx_v[r, pl.ds(gg * 16, 16)] = wh
            wrow_v[r, pl.ds(gg * 16, 16)] = lax.shift_right_logical(
                wh, jnp.full((16,), 3, jnp.int32))
        return jnp.int32(0)

    lax.fori_loop(jnp.int32(0), jnp.int32(_NCHUNK), hash_row, jnp.int32(0))

    copies = []
    for j in range(_NCHUNK):
        sl = pl.ds(j * _CHUNK, _CHUNK)
        ji = jnp.int32(j)
        copies.append(pltpu.async_copy(emb_hbm.at[idx0_v.at[ji]], e0_v.at[sl], sem))
        copies.append(pltpu.async_copy(emb_hbm.at[idx1_v.at[ji]], e1_v.at[sl], sem))
        copies.append(pltpu.async_copy(wemb_hbm.at[wrow_v.at[ji]], wg_v.at[sl], sem))
    for c in copies:
        c.wait()

    lnwa = lnw_v[pl.ds(0, 16)]
    lnwb = lnw_v[pl.ds(16, 16)]
    lnba = lnb_v[pl.ds(0, 16)]
    lnbb = lnb_v[pl.ds(16, 16)]
    zeros16 = jnp.zeros((16,), jnp.int32)
    ones16 = jnp.full((16,), 1, jnp.int32)

    def row_body(i, _):
        e0a = e0_v[i, pl.ds(0, 16)]
        e0b = e0_v[i, pl.ds(16, 16)]
        e1a = e1_v[i, pl.ds(0, 16)]
        e1b = e1_v[i, pl.ds(16, 16)]
        iv = jnp.full((16,), i, jnp.int32)
        ihi = lax.shift_right_logical(i, jnp.int32(7))
        ilo = jnp.bitwise_and(i, jnp.int32(127))
        wsp = plsc.load_gather(widx_v, [jnp.full((16,), ihi, jnp.int32),
                                        jnp.full((16,), ilo, jnp.int32)])
        col0 = jnp.bitwise_and(wsp * 2, 15)
        w0 = plsc.load_gather(wg_v, [iv, col0])
        w1 = plsc.load_gather(wg_v, [iv, col0 + 1])
        ra = e0a * w0 + e1a * w1
        rb = e0b * w0 + e1b * w1
        s = jnp.sum(ra + rb)
        s2 = jnp.sum(ra * ra + rb * rb)
        meanv = lax.broadcast(s, (16,)) * (1.0 / _DIM)
        ex2v = lax.broadcast(s2, (16,)) * (1.0 / _DIM)
        varv = ex2v - meanv * meanv + 1e-5
        bits = plsc.bitcast(varv, jnp.int32)
        bits = _MAGIC - lax.shift_right_logical(bits, jnp.full((16,), 1, jnp.int32))
        y = plsc.bitcast(bits, jnp.float32)
        half = varv * 0.5
        y = y * (1.5 - half * y * y)
        y = y * (1.5 - half * y * y)
        y = y * (1.5 - half * y * y)
        out_v[i, pl.ds(0, 16)] = (ra - meanv) * y * lnwa + lnba
        out_v[i, pl.ds(16, 16)] = (rb - meanv) * y * lnwb + lnbb
        return jnp.int32(0)

    lax.fori_loop(jnp.int32(0), jnp.int32(_BPW), row_body, jnp.int32(0))

    pltpu.sync_copy(out_v, out_hbm.at[pl.ds(base, _BPW)])


@functools.partial(jax.jit)
def _run(ids32, tabs, emb_table, weight_emb, ln_w, ln_b):
    mesh = plsc.VectorSubcoreMesh(core_axis_name="c", subcore_axis_name="s")
    kern = pl.kernel(
        _body,
        out_type=jax.ShapeDtypeStruct((_BATCH, _DIM), jnp.float32),
        mesh=mesh,
        scratch_types=[
            pltpu.VMEM((len(_HASH_SPECS) * 2048,), jnp.int32),
            pltpu.VMEM((_BPW,), jnp.int32),
            pltpu.VMEM((_NCHUNK, _CHUNK), jnp.int32),
            pltpu.VMEM((_NCHUNK, _CHUNK), jnp.int32),
            pltpu.VMEM((_NCHUNK, _CHUNK), jnp.int32),
            pltpu.VMEM((_NCHUNK, _CHUNK), jnp.int32),
            pltpu.VMEM((_BPW, _DIM), jnp.float32),
            pltpu.VMEM((_BPW, _DIM), jnp.float32),
            pltpu.VMEM((_BPW, 16), jnp.float32),
            pltpu.VMEM((_DIM,), jnp.float32),
            pltpu.VMEM((_DIM,), jnp.float32),
            pltpu.VMEM((_BPW, _DIM), jnp.float32),
            pltpu.SemaphoreType.DMA,
        ],
        compiler_params=pltpu.CompilerParams(
            needs_layout_passes=False, use_tc_tiling_on_sc=False),
    )
    return kern(ids32, tabs, emb_table, weight_emb, ln_w, ln_b)


def kernel(input_ids, emb_table, weight_emb, ln_w, ln_b):
    ids32 = input_ids.astype(jnp.int32)
    wemb16 = weight_emb.astype(jnp.float32).reshape(-1, 16)
    return _run(ids32, _TABLES, emb_table.astype(jnp.float32),
                wemb16, ln_w.astype(jnp.float32), ln_b.astype(jnp.float32))


# speedup vs baseline: 9.5894x; 9.5894x over previous
import functools
import math
import random as _random

import numpy as np
import jax
import jax.numpy as jnp
from jax import lax
from jax.experimental import pallas as pl
from jax.experimental.pallas import tpu as pltpu
from jax.experimental.pallas import tpu_sc as plsc

_NUM_HASHES = 2
_VOCAB = 1000000
_DIM = 32
_NUM_EMBEDS = 100000
_BATCH = 16384

_NC = 2
_NS = 16
_NW = _NC * _NS
_BPW = _BATCH // _NW
_CHUNK = 128
_NCHUNK = _BPW // _CHUNK

_MIN32 = -0x80000000
_MAGIC = 0x5F3759DF


def _draw_hash_constants():
    def is_prime(x):
        for i in range(2, int(math.sqrt(x))):
            if x % i == 0:
                return False
        return True

    def next_prime(n):
        while not is_prime(n):
            n += 1
        return n

    rng = _random.Random()
    rng.seed(1924031)

    def draw(N):
        p = next_prime(rng.randint(_VOCAB, int(2 ** 32)))
        a, b = rng.randint(1, p), rng.randint(1, p)
        return (a, b, p, N)

    hs = [draw(_NUM_EMBEDS) for _ in range(_NUM_HASHES)]
    wh = draw(_VOCAB)
    return hs + [wh]


_HASH_SPECS = _draw_hash_constants()


def _build_tables():
    tabs = np.zeros((len(_HASH_SPECS) * 2048,), dtype=np.uint32)
    for h, (a, b, p, _N) in enumerate(_HASH_SPECS):
        j = np.arange(1024, dtype=object)
        t1 = np.array([(a * 1024 * int(x)) % p for x in range(1024)], dtype=np.uint64)
        t0 = np.array([(a * int(x) + b) % p for x in range(1024)], dtype=np.uint64)
        tabs[h * 2048: h * 2048 + 1024] = t1.astype(np.uint32)
        tabs[h * 2048 + 1024: h * 2048 + 2048] = t0.astype(np.uint32)
    return jnp.asarray(tabs.view(np.int32))


_TABLES = _build_tables()


def _i32(v):
    return np.int32(np.uint32(v & 0xFFFFFFFF)).item()


def _hash_mod(tabs_ref, x, h):
    a, b, p, N = _HASH_SPECS[h]
    off = h * 2048
    x1 = lax.shift_right_logical(x, jnp.full(x.shape, 10, jnp.int32))
    x0 = jnp.bitwise_and(x, 1023)
    t1 = plsc.load_gather(tabs_ref, [x1 + off])
    t0 = plsc.load_gather(tabs_ref, [x0 + (off + 1024)])
    s = t1 + t0
    sm = jnp.bitwise_xor(s, _MIN32)
    t1m = jnp.bitwise_xor(t1, _MIN32)
    cond = jnp.logical_or(sm < t1m, sm >= _i32(p ^ 0x80000000))
    r = jnp.where(cond, s - _i32(p), s)
    hi = lax.shift_right_logical(r, jnp.full(r.shape, 8, jnp.int32))
    lo = jnp.bitwise_and(r, 255)
    rf = lax.convert_element_type(hi, jnp.float32) * 256.0 + \
        lax.convert_element_type(lo, jnp.float32)
    q = lax.convert_element_type(rf * (1.0 / N), jnp.int32)
    rem = r - q * N
    rem = jnp.where(rem < 0, rem + N, rem)
    rem = jnp.where(rem >= N, rem - N, rem)
    return rem


def _body(ids_hbm, tabs_hbm, emb_hbm, w0_hbm, w1_hbm, lnw_hbm, lnb_hbm, out_hbm,
          tabs_v, ids_v, idx0_v, idx1_v, widx_v, wrow_v, e0_v, e1_v,
          w0g_v, w1g_v, lnw_v, lnb_v, out_v, sem):
    wid = lax.axis_index("s") * jnp.int32(_NC) + lax.axis_index("c")
    base = wid * jnp.int32(_BPW)

    pltpu.sync_copy(ids_hbm.at[pl.ds(base, _BPW)], ids_v)
    pltpu.sync_copy(tabs_hbm, tabs_v)
    pltpu.sync_copy(lnw_hbm, lnw_v)
    pltpu.sync_copy(lnb_hbm, lnb_v)

    def hash_row(r, _):
        for gg in range(_CHUNK // 16):
            x = ids_v[pl.ds(r * jnp.int32(_CHUNK) + jnp.int32(gg * 16), 16)]
            idx0_v[r, pl.ds(gg * 16, 16)] = _hash_mod(tabs_v, x, 0)
            idx1_v[r, pl.ds(gg * 16, 16)] = _hash_mod(tabs_v, x, 1)
            wh = _hash_mod(tabs_v, x, 2)
            widx_v[r, pl.ds(gg * 16, 16)] = wh
            wrow_v[r, pl.ds(gg * 16, 16)] = lax.shift_right_logical(
                wh, jnp.full((16,), 4, jnp.int32))
        return jnp.int32(0)

    lax.fori_loop(jnp.int32(0), jnp.int32(_NCHUNK), hash_row, jnp.int32(0))

    copies = []
    for j in range(_NCHUNK):
        sl = pl.ds(j * _CHUNK, _CHUNK)
        ji = jnp.int32(j)
        copies.append(pltpu.async_copy(emb_hbm.at[idx0_v.at[ji]], e0_v.at[sl], sem))
        copies.append(pltpu.async_copy(emb_hbm.at[idx1_v.at[ji]], e1_v.at[sl], sem))
        copies.append(pltpu.async_copy(w0_hbm.at[wrow_v.at[ji]], w0g_v.at[sl], sem))
        copies.append(pltpu.async_copy(w1_hbm.at[wrow_v.at[ji]], w1g_v.at[sl], sem))
    for c in copies:
        c.wait()

    lnwa = lnw_v[pl.ds(0, 16)]
    lnwb = lnw_v[pl.ds(16, 16)]
    lnba = lnb_v[pl.ds(0, 16)]
    lnbb = lnb_v[pl.ds(16, 16)]
    zeros16 = jnp.zeros((16,), jnp.int32)
    ones16 = jnp.full((16,), 1, jnp.int32)

    def row_body(i, _):
        e0a = e0_v[i, pl.ds(0, 16)]
        e0b = e0_v[i, pl.ds(16, 16)]
        e1a = e1_v[i, pl.ds(0, 16)]
        e1b = e1_v[i, pl.ds(16, 16)]
        iv = jnp.full((16,), i, jnp.int32)
        ihi = lax.shift_right_logical(i, jnp.int32(7))
        ilo = jnp.bitwise_and(i, jnp.int32(127))
        wsp = plsc.load_gather(widx_v, [jnp.full((16,), ihi, jnp.int32),
                                        jnp.full((16,), ilo, jnp.int32)])
        col = jnp.bitwise_and(wsp, 15)
        w0 = plsc.load_gather(w0g_v, [iv, col])
        w1 = plsc.load_gather(w1g_v, [iv, col])
        ra = e0a * w0 + e1a * w1
        rb = e0b * w0 + e1b * w1
        s = jnp.sum(ra + rb)
        s2 = jnp.sum(ra * ra + rb * rb)
        meanv = lax.broadcast(s, (16,)) * (1.0 / _DIM)
        ex2v = lax.broadcast(s2, (16,)) * (1.0 / _DIM)
        varv = ex2v - meanv * meanv + 1e-5
        bits = plsc.bitcast(varv, jnp.int32)
        bits = _MAGIC - lax.shift_right_logical(bits, jnp.full((16,), 1, jnp.int32))
        y = plsc.bitcast(bits, jnp.float32)
        half = varv * 0.5
        y = y * (1.5 - half * y * y)
        y = y * (1.5 - half * y * y)
        y = y * (1.5 - half * y * y)
        out_v[i, pl.ds(0, 16)] = (ra - meanv) * y * lnwa + lnba
        out_v[i, pl.ds(16, 16)] = (rb - meanv) * y * lnwb + lnbb
        return jnp.int32(0)

    lax.fori_loop(jnp.int32(0), jnp.int32(_BPW), row_body, jnp.int32(0))

    pltpu.sync_copy(out_v, out_hbm.at[pl.ds(base, _BPW)])


@functools.partial(jax.jit)
def _run(ids32, tabs, emb_table, w0, w1, ln_w, ln_b):
    mesh = plsc.VectorSubcoreMesh(core_axis_name="c", subcore_axis_name="s")
    kern = pl.kernel(
        _body,
        out_type=jax.ShapeDtypeStruct((_BATCH, _DIM), jnp.float32),
        mesh=mesh,
        scratch_types=[
            pltpu.VMEM((len(_HASH_SPECS) * 2048,), jnp.int32),
            pltpu.VMEM((_BPW,), jnp.int32),
            pltpu.VMEM((_NCHUNK, _CHUNK), jnp.int32),
            pltpu.VMEM((_NCHUNK, _CHUNK), jnp.int32),
            pltpu.VMEM((_NCHUNK, _CHUNK), jnp.int32),
            pltpu.VMEM((_NCHUNK, _CHUNK), jnp.int32),
            pltpu.VMEM((_BPW, _DIM), jnp.float32),
            pltpu.VMEM((_BPW, _DIM), jnp.float32),
            pltpu.VMEM((_BPW, 16), jnp.float32),
            pltpu.VMEM((_BPW, 16), jnp.float32),
            pltpu.VMEM((_DIM,), jnp.float32),
            pltpu.VMEM((_DIM,), jnp.float32),
            pltpu.VMEM((_BPW, _DIM), jnp.float32),
            pltpu.SemaphoreType.DMA,
        ],
        compiler_params=pltpu.CompilerParams(
            needs_layout_passes=False, use_tc_tiling_on_sc=False),
    )
    return kern(ids32, tabs, emb_table, w0, w1, ln_w, ln_b)


def kernel(input_ids, emb_table, weight_emb, ln_w, ln_b):
    ids32 = input_ids.astype(jnp.int32)
    w0 = weight_emb[:, 0].reshape(-1, 16)
    w1 = weight_emb[:, 1].reshape(-1, 16)
    return _run(ids32, _TABLES, emb_table.astype(jnp.float32),
                w0, w1, ln_w.astype(jnp.float32), ln_b.astype(jnp.float32))


# speedup vs baseline: 11.8462x; 1.2353x over previous
import functools
import math
import random as _random

import numpy as np
import jax
import jax.numpy as jnp
from jax import lax
from jax.experimental import pallas as pl
from jax.experimental.pallas import tpu as pltpu
from jax.experimental.pallas import tpu_sc as plsc

_NUM_HASHES = 2
_VOCAB = 1000000
_DIM = 32
_NUM_EMBEDS = 100000
_BATCH = 16384

_NC = 2
_NS = 16
_NW = _NC * _NS
_BPW = _BATCH // _NW
_CHUNK = 128
_NCHUNK = _BPW // _CHUNK
_WROWS = _VOCAB // 16

_MIN32 = -0x80000000
_MAGIC = 0x5F3759DF


def _draw_hash_constants():
    def is_prime(x):
        for i in range(2, int(math.sqrt(x))):
            if x % i == 0:
                return False
        return True

    def next_prime(n):
        while not is_prime(n):
            n += 1
        return n

    rng = _random.Random()
    rng.seed(1924031)

    def draw(N):
        p = next_prime(rng.randint(_VOCAB, int(2 ** 32)))
        a, b = rng.randint(1, p), rng.randint(1, p)
        return (a, b, p, N)

    hs = [draw(_NUM_EMBEDS) for _ in range(_NUM_HASHES)]
    wh = draw(_VOCAB)
    return hs + [wh]


_HASH_SPECS = _draw_hash_constants()


def _build_tables():
    tabs = np.zeros((len(_HASH_SPECS) * 2048,), dtype=np.uint32)
    for h, (a, b, p, _N) in enumerate(_HASH_SPECS):
        t1 = np.array([(a * 1024 * int(x)) % p for x in range(1024)], dtype=np.uint64)
        t0 = np.array([(a * int(x) + b) % p for x in range(1024)], dtype=np.uint64)
        tabs[h * 2048: h * 2048 + 1024] = t1.astype(np.uint32)
        tabs[h * 2048 + 1024: h * 2048 + 2048] = t0.astype(np.uint32)
    return jnp.asarray(tabs.view(np.int32))


_TABLES = _build_tables()


def _i32(v):
    return np.int32(np.uint32(v & 0xFFFFFFFF)).item()


def _hash_mod(tabs_ref, x, h):
    a, b, p, N = _HASH_SPECS[h]
    off = h * 2048
    x1 = lax.shift_right_logical(x, jnp.full(x.shape, 10, jnp.int32))
    x0 = jnp.bitwise_and(x, 1023)
    t1 = plsc.load_gather(tabs_ref, [x1 + off])
    t0 = plsc.load_gather(tabs_ref, [x0 + (off + 1024)])
    s = t1 + t0
    sm = jnp.bitwise_xor(s, _MIN32)
    t1m = jnp.bitwise_xor(t1, _MIN32)
    cond = jnp.logical_or(sm < t1m, sm >= _i32(p ^ 0x80000000))
    r = jnp.where(cond, s - _i32(p), s)
    hi = lax.shift_right_logical(r, jnp.full(r.shape, 8, jnp.int32))
    lo = jnp.bitwise_and(r, 255)
    rf = lax.convert_element_type(hi, jnp.float32) * 256.0 + \
        lax.convert_element_type(lo, jnp.float32)
    q = lax.convert_element_type(rf * (1.0 / N), jnp.int32)
    rem = r - q * N
    rem = jnp.where(rem < 0, rem + N, rem)
    rem = jnp.where(rem >= N, rem - N, rem)
    return rem


def _newton_rsqrt(x):
    bits = plsc.bitcast(x, jnp.int32)
    bits = _MAGIC - lax.shift_right_logical(bits, jnp.full(x.shape, 1, jnp.int32))
    y = plsc.bitcast(bits, jnp.float32)
    half = x * 0.5
    y = y * (1.5 - half * y * y)
    y = y * (1.5 - half * y * y)
    y = y * (1.5 - half * y * y)
    return y


def _body(ids_hbm, tabs_hbm, emb_hbm, wp_hbm, lnw_hbm, lnb_hbm, out_hbm,
          tabs_v, ids_v, idx0_v, idx1_v, widx_v, wrow_v, wrow1_v,
          e0_v, e1_v, w0g_v, w1g_v, lnw_v, lnb_v, lnsp_v,
          mb_v, rb_v, outT_v, sem0, sem1, sem2, sem3):
    wid = lax.axis_index("s") * jnp.int32(_NC) + lax.axis_index("c")
    base = wid * jnp.int32(_BPW)
    sems = [sem0, sem1, sem2, sem3]

    pltpu.sync_copy(ids_hbm.at[pl.ds(base, _BPW)], ids_v)
    pltpu.sync_copy(tabs_hbm, tabs_v)
    pltpu.sync_copy(lnw_hbm, lnw_v)
    pltpu.sync_copy(lnb_hbm, lnb_v)

    lane = lax.iota(jnp.int32, 16)

    def hash_row(r, _):
        for gg in range(_CHUNK // 16):
            x = ids_v[pl.ds(r * jnp.int32(_CHUNK) + jnp.int32(gg * 16), 16)]
            idx0_v[r, pl.ds(gg * 16, 16)] = _hash_mod(tabs_v, x, 0)
            idx1_v[r, pl.ds(gg * 16, 16)] = _hash_mod(tabs_v, x, 1)
            wh = _hash_mod(tabs_v, x, 2)
            widx_v[r, pl.ds(gg * 16, 16)] = wh
            wr = lax.shift_right_logical(wh, jnp.full((16,), 4, jnp.int32))
            wrow_v[r, pl.ds(gg * 16, 16)] = wr
            wrow1_v[r, pl.ds(gg * 16, 16)] = wr + jnp.int32(_WROWS)
        return jnp.int32(0)

    lax.fori_loop(jnp.int32(0), jnp.int32(_NCHUNK), hash_row, jnp.int32(0))

    chunk_copies = []
    for j in range(_NCHUNK):
        sl = pl.ds(j * _CHUNK, _CHUNK)
        ji = jnp.int32(j)
        cs = [
            pltpu.async_copy(emb_hbm.at[idx0_v.at[ji]], e0_v.at[sl], sems[j]),
            pltpu.async_copy(emb_hbm.at[idx1_v.at[ji]], e1_v.at[sl], sems[j]),
            pltpu.async_copy(wp_hbm.at[wrow_v.at[ji]], w0g_v.at[sl], sems[j]),
            pltpu.async_copy(wp_hbm.at[wrow1_v.at[ji]], w1g_v.at[sl], sems[j]),
        ]
        chunk_copies.append(cs)

    for d in range(_DIM):
        dv = jnp.full((16,), d, jnp.int32)
        lnsp_v[0, jnp.int32(d)] = plsc.load_gather(lnw_v, [dv])
        lnsp_v[1, jnp.int32(d)] = plsc.load_gather(lnb_v, [dv])

    inv_dim = 1.0 / _DIM

    for j in range(_NCHUNK):
        for c in chunk_copies[j]:
            c.wait()

        def pass1(g, _, j=j):
            col = jnp.int32(j * _CHUNK) + g * jnp.int32(16)
            rows = lane + col
            widxv = widx_v[jnp.int32(j), pl.ds(g * jnp.int32(16), 16)]
            colv = jnp.bitwise_and(widxv, 15)
            w0 = plsc.load_gather(w0g_v, [rows, colv])
            w1 = plsc.load_gather(w1g_v, [rows, colv])
            s = jnp.zeros((16,), jnp.float32)
            s2 = jnp.zeros((16,), jnp.float32)
            for d in range(_DIM):
                dv = jnp.full((16,), d, jnp.int32)
                e0d = plsc.load_gather(e0_v, [rows, dv])
                e1d = plsc.load_gather(e1_v, [rows, dv])
                a = e0d * w0 + e1d * w1
                outT_v[jnp.int32(d), pl.ds(col, 16)] = a
                s = s + a
                s2 = s2 + a * a
            meanv = s * inv_dim
            varv = s2 * inv_dim - meanv * meanv + 1e-5
            mb_v[pl.ds(col, 16)] = meanv
            rb_v[pl.ds(col, 16)] = _newton_rsqrt(varv)
            return jnp.int32(0)

        lax.fori_loop(jnp.int32(0), jnp.int32(_CHUNK // 16), pass1, jnp.int32(0))

        def pass2(g, _, j=j):
            col = jnp.int32(j * _CHUNK) + g * jnp.int32(16)
            m = mb_v[pl.ds(col, 16)]
            r = rb_v[pl.ds(col, 16)]
            for d in range(_DIM):
                a = outT_v[jnp.int32(d), pl.ds(col, 16)]
                o = (a - m) * r * lnsp_v[0, jnp.int32(d)] + lnsp_v[1, jnp.int32(d)]
                outT_v[jnp.int32(d), pl.ds(col, 16)] = o
            return jnp.int32(0)

        lax.fori_loop(jnp.int32(0), jnp.int32(_CHUNK // 16), pass2, jnp.int32(0))

    pltpu.sync_copy(outT_v, out_hbm.at[:, pl.ds(base, _BPW)])


@functools.partial(jax.jit)
def _run(ids32, tabs, emb_table, wplanes, ln_w, ln_b):
    mesh = plsc.VectorSubcoreMesh(core_axis_name="c", subcore_axis_name="s")
    kern = pl.kernel(
        _body,
        out_type=jax.ShapeDtypeStruct((_DIM, _BATCH), jnp.float32),
        mesh=mesh,
        scratch_types=[
            pltpu.VMEM((len(_HASH_SPECS) * 2048,), jnp.int32),
            pltpu.VMEM((_BPW,), jnp.int32),
            pltpu.VMEM((_NCHUNK, _CHUNK), jnp.int32),
            pltpu.VMEM((_NCHUNK, _CHUNK), jnp.int32),
            pltpu.VMEM((_NCHUNK, _CHUNK), jnp.int32),
            pltpu.VMEM((_NCHUNK, _CHUNK), jnp.int32),
            pltpu.VMEM((_NCHUNK, _CHUNK), jnp.int32),
            pltpu.VMEM((_BPW, _DIM), jnp.float32),
            pltpu.VMEM((_BPW, _DIM), jnp.float32),
            pltpu.VMEM((_BPW, 16), jnp.float32),
            pltpu.VMEM((_BPW, 16), jnp.float32),
            pltpu.VMEM((_DIM,), jnp.float32),
            pltpu.VMEM((_DIM,), jnp.float32),
            pltpu.VMEM((2, _DIM, 16), jnp.float32),
            pltpu.VMEM((_BPW,), jnp.float32),
            pltpu.VMEM((_BPW,), jnp.float32),
            pltpu.VMEM((_DIM, _BPW), jnp.float32),
            pltpu.SemaphoreType.DMA,
            pltpu.SemaphoreType.DMA,
            pltpu.SemaphoreType.DMA,
            pltpu.SemaphoreType.DMA,
        ],
        compiler_params=pltpu.CompilerParams(
            needs_layout_passes=False, use_tc_tiling_on_sc=False),
    )
    outT = kern(ids32, tabs, emb_table, wplanes, ln_w, ln_b)
    return outT.T


def kernel(input_ids, emb_table, weight_emb, ln_w, ln_b):
    ids32 = input_ids.astype(jnp.int32)
    wplanes = weight_emb.T.reshape(2 * _WROWS, 16)
    return _run(ids32, _TABLES, emb_table.astype(jnp.float32),
                wplanes, ln_w.astype(jnp.float32), ln_b.astype(jnp.float32))


# speedup vs baseline: 12.7483x; 1.0762x over previous
import functools
import math
import random as _random

import numpy as np
import jax
import jax.numpy as jnp
from jax import lax
from jax.experimental import pallas as pl
from jax.experimental.pallas import tpu as pltpu
from jax.experimental.pallas import tpu_sc as plsc

_NUM_HASHES = 2
_VOCAB = 1000000
_DIM = 32
_NUM_EMBEDS = 100000
_BATCH = 16384

_NC = 2
_NS = 16
_NW = _NC * _NS
_BPW = _BATCH // _NW
_CHUNK = 128
_NCHUNK = _BPW // _CHUNK
_WROWS = _VOCAB // 16

_MIN32 = -0x80000000
_MAGIC = 0x5F3759DF


def _draw_hash_constants():
    def is_prime(x):
        for i in range(2, int(math.sqrt(x))):
            if x % i == 0:
                return False
        return True

    def next_prime(n):
        while not is_prime(n):
            n += 1
        return n

    rng = _random.Random()
    rng.seed(1924031)

    def draw(N):
        p = next_prime(rng.randint(_VOCAB, int(2 ** 32)))
        a, b = rng.randint(1, p), rng.randint(1, p)
        return (a, b, p, N)

    hs = [draw(_NUM_EMBEDS) for _ in range(_NUM_HASHES)]
    wh = draw(_VOCAB)
    return hs + [wh]


_HASH_SPECS = _draw_hash_constants()


def _build_tables():
    tabs = np.zeros((len(_HASH_SPECS) * 2048,), dtype=np.uint32)
    for h, (a, b, p, _N) in enumerate(_HASH_SPECS):
        t1 = np.array([(a * 1024 * int(x)) % p for x in range(1024)], dtype=np.uint64)
        t0 = np.array([(a * int(x) + b) % p for x in range(1024)], dtype=np.uint64)
        tabs[h * 2048: h * 2048 + 1024] = t1.astype(np.uint32)
        tabs[h * 2048 + 1024: h * 2048 + 2048] = t0.astype(np.uint32)
    return jnp.asarray(tabs.view(np.int32))


_TABLES = _build_tables()


def _i32(v):
    return np.int32(np.uint32(v & 0xFFFFFFFF)).item()


def _hash_mod(tabs_ref, x, h):
    a, b, p, N = _HASH_SPECS[h]
    off = h * 2048
    x1 = lax.shift_right_logical(x, jnp.full(x.shape, 10, jnp.int32))
    x0 = jnp.bitwise_and(x, 1023)
    t1 = plsc.load_gather(tabs_ref, [x1 + off])
    t0 = plsc.load_gather(tabs_ref, [x0 + (off + 1024)])
    s = t1 + t0
    sm = jnp.bitwise_xor(s, _MIN32)
    t1m = jnp.bitwise_xor(t1, _MIN32)
    cond = jnp.logical_or(sm < t1m, sm >= _i32(p ^ 0x80000000))
    r = jnp.where(cond, s - _i32(p), s)
    hi = lax.shift_right_logical(r, jnp.full(r.shape, 8, jnp.int32))
    lo = jnp.bitwise_and(r, 255)
    rf = lax.convert_element_type(hi, jnp.float32) * 256.0 + \
        lax.convert_element_type(lo, jnp.float32)
    q = lax.convert_element_type(rf * (1.0 / N), jnp.int32)
    rem = r - q * N
    rem = jnp.where(rem < 0, rem + N, rem)
    rem = jnp.where(rem >= N, rem - N, rem)
    return rem


def _newton_rsqrt(x):
    bits = plsc.bitcast(x, jnp.int32)
    bits = _MAGIC - lax.shift_right_logical(bits, jnp.full(x.shape, 1, jnp.int32))
    y = plsc.bitcast(bits, jnp.float32)
    half = x * 0.5
    y = y * (1.5 - half * y * y)
    y = y * (1.5 - half * y * y)
    y = y * (1.5 - half * y * y)
    return y


def _body(ids_hbm, tabs_hbm, emb_hbm, wp_hbm, lnw_hbm, lnb_hbm, out_hbm,
          tabs_v, ids_v, idx0_v, idx1_v, widx_v, wrow_v, wrow1_v,
          e0_v, e1_v, w0g_v, w1g_v, lnw_v, lnb_v, lnsp_v,
          mb_v, rb_v, outT_v, sem0, sem1, sem2, sem3):
    wid = lax.axis_index("s") * jnp.int32(_NC) + lax.axis_index("c")
    base = wid * jnp.int32(_BPW)
    sems = [sem0, sem1, sem2, sem3]

    pltpu.sync_copy(ids_hbm.at[pl.ds(base, _BPW)], ids_v)
    pltpu.sync_copy(tabs_hbm, tabs_v)
    pltpu.sync_copy(lnw_hbm, lnw_v)
    pltpu.sync_copy(lnb_hbm, lnb_v)

    lane = lax.iota(jnp.int32, 16)

    @plsc.parallel_loop(jnp.int32(0), jnp.int32(_NCHUNK), jnp.int32(1), unroll=2)
    def hash_row(r):
        for gg in range(_CHUNK // 16):
            x = ids_v[pl.ds(r * jnp.int32(_CHUNK) + jnp.int32(gg * 16), 16)]
            idx0_v[r, pl.ds(gg * 16, 16)] = _hash_mod(tabs_v, x, 0)
            idx1_v[r, pl.ds(gg * 16, 16)] = _hash_mod(tabs_v, x, 1)
            wh = _hash_mod(tabs_v, x, 2)
            widx_v[r, pl.ds(gg * 16, 16)] = wh
            wr = lax.shift_right_logical(wh, jnp.full((16,), 4, jnp.int32))
            wrow_v[r, pl.ds(gg * 16, 16)] = wr
            wrow1_v[r, pl.ds(gg * 16, 16)] = wr + jnp.int32(_WROWS)

    chunk_copies = []
    for j in range(_NCHUNK):
        sl = pl.ds(j * _CHUNK, _CHUNK)
        ji = jnp.int32(j)
        cs = [
            pltpu.async_copy(emb_hbm.at[idx0_v.at[ji]], e0_v.at[sl], sems[j]),
            pltpu.async_copy(emb_hbm.at[idx1_v.at[ji]], e1_v.at[sl], sems[j]),
            pltpu.async_copy(wp_hbm.at[wrow_v.at[ji]], w0g_v.at[sl], sems[j]),
            pltpu.async_copy(wp_hbm.at[wrow1_v.at[ji]], w1g_v.at[sl], sems[j]),
        ]
        chunk_copies.append(cs)

    for d in range(_DIM):
        dv = jnp.full((16,), d, jnp.int32)
        lnsp_v[0, jnp.int32(d)] = plsc.load_gather(lnw_v, [dv])
        lnsp_v[1, jnp.int32(d)] = plsc.load_gather(lnb_v, [dv])

    inv_dim = 1.0 / _DIM

    for j in range(_NCHUNK):
        for c in chunk_copies[j]:
            c.wait()

        @plsc.parallel_loop(jnp.int32(0), jnp.int32(_CHUNK // 16), jnp.int32(1), unroll=2)
        def group_body(g, j=j):
            col = jnp.int32(j * _CHUNK) + g * jnp.int32(16)
            rows = lane + col
            widxv = widx_v[jnp.int32(j), pl.ds(g * jnp.int32(16), 16)]
            colv = jnp.bitwise_and(widxv, 15)
            w0 = plsc.load_gather(w0g_v, [rows, colv])
            w1 = plsc.load_gather(w1g_v, [rows, colv])
            s = jnp.zeros((16,), jnp.float32)
            s2 = jnp.zeros((16,), jnp.float32)
            for d in range(_DIM):
                dv = jnp.full((16,), d, jnp.int32)
                e0d = plsc.load_gather(e0_v, [rows, dv])
                e1d = plsc.load_gather(e1_v, [rows, dv])
                a = e0d * w0 + e1d * w1
                outT_v[jnp.int32(d), pl.ds(col, 16)] = a
                s = s + a
                s2 = s2 + a * a
            meanv = s * inv_dim
            varv = s2 * inv_dim - meanv * meanv + 1e-5
            rstd = _newton_rsqrt(varv)
            for d in range(_DIM):
                a = outT_v[jnp.int32(d), pl.ds(col, 16)]
                o = (a - meanv) * rstd * lnsp_v[0, jnp.int32(d)] + \
                    lnsp_v[1, jnp.int32(d)]
                outT_v[jnp.int32(d), pl.ds(col, 16)] = o

    pltpu.sync_copy(outT_v, out_hbm.at[:, pl.ds(base, _BPW)])


@functools.partial(jax.jit)
def _run(ids32, tabs, emb_table, wplanes, ln_w, ln_b):
    mesh = plsc.VectorSubcoreMesh(core_axis_name="c", subcore_axis_name="s")
    kern = pl.kernel(
        _body,
        out_type=jax.ShapeDtypeStruct((_DIM, _BATCH), jnp.float32),
        mesh=mesh,
        scratch_types=[
            pltpu.VMEM((len(_HASH_SPECS) * 2048,), jnp.int32),
            pltpu.VMEM((_BPW,), jnp.int32),
            pltpu.VMEM((_NCHUNK, _CHUNK), jnp.int32),
            pltpu.VMEM((_NCHUNK, _CHUNK), jnp.int32),
            pltpu.VMEM((_NCHUNK, _CHUNK), jnp.int32),
            pltpu.VMEM((_NCHUNK, _CHUNK), jnp.int32),
            pltpu.VMEM((_NCHUNK, _CHUNK), jnp.int32),
            pltpu.VMEM((_BPW, _DIM), jnp.float32),
            pltpu.VMEM((_BPW, _DIM), jnp.float32),
            pltpu.VMEM((_BPW, 16), jnp.float32),
            pltpu.VMEM((_BPW, 16), jnp.float32),
            pltpu.VMEM((_DIM,), jnp.float32),
            pltpu.VMEM((_DIM,), jnp.float32),
            pltpu.VMEM((2, _DIM, 16), jnp.float32),
            pltpu.VMEM((_BPW,), jnp.float32),
            pltpu.VMEM((_BPW,), jnp.float32),
            pltpu.VMEM((_DIM, _BPW), jnp.float32),
            pltpu.SemaphoreType.DMA,
            pltpu.SemaphoreType.DMA,
            pltpu.SemaphoreType.DMA,
            pltpu.SemaphoreType.DMA,
        ],
        compiler_params=pltpu.CompilerParams(
            needs_layout_passes=False, use_tc_tiling_on_sc=False),
    )
    outT = kern(ids32, tabs, emb_table, wplanes, ln_w, ln_b)
    return outT.T


def kernel(input_ids, emb_table, weight_emb, ln_w, ln_b):
    ids32 = input_ids.astype(jnp.int32)
    wplanes = weight_emb.T.reshape(2 * _WROWS, 16)
    return _run(ids32, _TABLES, emb_table.astype(jnp.float32),
                wplanes, ln_w.astype(jnp.float32), ln_b.astype(jnp.float32))
